# baseline (speedup 1.0000x reference)
import jax
import jax.numpy as jnp
import numpy as np
from jax.experimental import pallas as pl

_T = 128
_H = 64
_D = 64
_TOPK = 8
_MAXR = 255.0

_BB = 64


def _make_d8m():
    i = np.arange(_T)
    d = np.abs(i[:, None] - i[None, :]).astype(np.float32)
    decay = np.float32(1.0) - (np.float32(0.1) * d) / np.float32(128.0)
    tril = i[None, :] <= i[:, None]
    return np.where(tril, decay * np.float32(0.125),
                    np.float32(0.0)).astype(np.float32)


_D8M = _make_d8m()


def _row_quant(f_parts, bdims):
    sum_ = None
    sumsq = None
    thresh = None
    for fp in f_parts:
        s1 = jnp.sum(fp, axis=-1, keepdims=True)
        s2 = jnp.sum(fp * fp, axis=-1, keepdims=True)
        t1 = jnp.max(fp, axis=-1, keepdims=True)
        sum_ = s1 if sum_ is None else sum_ + s1
        sumsq = s2 if sumsq is None else sumsq + s2
        thresh = t1 if thresh is None else jnp.maximum(thresh, t1)
    m = thresh
    mean = sum_ * (1.0 / _T)
    var = jnp.maximum(sumsq - mean * mean * _T, 0.0) / (_T - 1)
    sigma = jnp.sqrt(var)

    for _ in range(_TOPK - 1):
        nxt = None
        for fp in f_parts:
            t1 = jnp.max(jnp.where(fp >= thresh, -1.0, fp),
                         axis=-1, keepdims=True)
            nxt = t1 if nxt is None else jnp.maximum(nxt, t1)
        thresh = nxt

    r = _MAXR / (jnp.maximum(m, sigma) + 1e-6)
    out = []
    for fp in f_parts:
        norm = jnp.floor(fp * r)
        out.append(jnp.where(
            fp >= thresh, norm - jnp.where(norm > 127.5, 256.0, 0.0), 0.0))
    return out


def _head_body(x_ref, wq_ref, wk_ref, wv_ref, d8m_ref, g_ref, out_ref):
    x = x_ref[...].reshape(_BB * _T, _D)
    q = jnp.dot(x, wq_ref[...], preferred_element_type=jnp.float32)
    k = jnp.dot(x, wk_ref[...], preferred_element_type=jnp.float32)
    v = jnp.dot(x, wv_ref[...], preferred_element_type=jnp.float32)
    q = q.reshape(_BB, _T, _D)
    k = k.reshape(_BB, _T, _D)
    v = v.reshape(_BB, _T, _D) * (1.0 / g_ref[0, 0])

    qa, qb = q[:, :_H], q[:, _H:]
    ka, kb = k[:, :_H], k[:, _H:]
    va, vb = v[:, :_H], v[:, _H:]

    dn = (((2,), (2,)), ((0,), (0,)))
    s_aa = jax.lax.dot_general(qa, ka, dn, preferred_element_type=jnp.float32)
    s_ba = jax.lax.dot_general(qb, ka, dn, preferred_element_type=jnp.float32)
    s_bb = jax.lax.dot_general(qb, kb, dn, preferred_element_type=jnp.float32)

    d_aa = d8m_ref[:_H, :_H][None]
    d_ba = d8m_ref[_H:, :_H][None]
    d_bb = d8m_ref[_H:, _H:][None]
    f_aa = jnp.maximum(s_aa, 0.0) * d_aa
    f_ba = jnp.maximum(s_ba, 0.0) * d_ba
    f_bb = jnp.maximum(s_bb, 0.0) * d_bb

    (w_aa,) = _row_quant([f_aa], _BB)
    w_ba, w_bb = _row_quant([f_ba, f_bb], _BB)

    dw = (((2,), (1,)), ((0,), (0,)))
    out_a = jax.lax.dot_general(w_aa, va, dw,
                                preferred_element_type=jnp.float32)
    out_b = (jax.lax.dot_general(w_ba, va, dw,
                                 preferred_element_type=jnp.float32)
             + jax.lax.dot_general(w_bb, vb, dw,
                                   preferred_element_type=jnp.float32))

    out_ref[:, :_H] = out_a
    out_ref[:, _H:] = out_b


def kernel(x, Wk, Wq, Wv, gamma):
    b, t, d = x.shape
    g = jnp.reshape(gamma, (1, 1)).astype(jnp.float32)
    return pl.pallas_call(
        _head_body,
        grid=(b // _BB,),
        in_specs=[
            pl.BlockSpec((_BB, t, d), lambda i: (i, 0, 0)),
            pl.BlockSpec((d, d), lambda i: (0, 0)),
            pl.BlockSpec((d, d), lambda i: (0, 0)),
            pl.BlockSpec((d, d), lambda i: (0, 0)),
            pl.BlockSpec((t, t), lambda i: (0, 0)),
            pl.BlockSpec((1, 1), lambda i: (0, 0)),
        ],
        out_specs=pl.BlockSpec((_BB, t, d), lambda i: (i, 0, 0)),
        out_shape=jax.ShapeDtypeStruct((b, t, d), jnp.float32),
    )(x, Wq, Wk, Wv, jnp.asarray(_D8M), g)


# speedup vs baseline: 1.2886x; 1.2886x over previous
import jax
import jax.numpy as jnp
import numpy as np
from jax.experimental import pallas as pl

_T = 128
_D = 64
_TOPK = 8
_MAXR = 255.0

_BB = 64


def _make_d8m():
    i = np.arange(_T)
    d = np.abs(i[:, None] - i[None, :]).astype(np.float32)
    decay = np.float32(1.0) - (np.float32(0.1) * d) / np.float32(128.0)
    tril = i[None, :] <= i[:, None]
    return np.where(tril, decay * np.float32(0.125),
                    np.float32(0.0)).astype(np.float32)


_D8M = _make_d8m()


def _head_body(x_ref, wq_ref, wk_ref, wv_ref, d8m_ref, g_ref, out_ref):
    x = x_ref[...].reshape(_BB * _T, _D)
    q = jnp.dot(x, wq_ref[...], preferred_element_type=jnp.float32)
    k = jnp.dot(x, wk_ref[...], preferred_element_type=jnp.float32)
    v = jnp.dot(x, wv_ref[...], preferred_element_type=jnp.float32)
    q = q.reshape(_BB, _T, _D)
    k = k.reshape(_BB, _T, _D)
    v = v.reshape(_BB, _T, _D) * (1.0 / g_ref[0, 0])

    s = jax.lax.dot_general(
        q, k, (((2,), (2,)), ((0,), (0,))),
        preferred_element_type=jnp.float32)

    f = jnp.maximum(s, 0.0) * d8m_ref[...][None]

    mean = jnp.mean(f, axis=-1, keepdims=True)
    sumsq = jnp.sum(f * f, axis=-1, keepdims=True)
    var = jnp.maximum(sumsq - mean * mean * _T, 0.0) / (_T - 1)
    sigma = jnp.sqrt(var)

    thresh = jnp.max(f, axis=-1, keepdims=True)
    m = thresh
    for _ in range(_TOPK - 1):
        thresh = jnp.max(f * (f < thresh), axis=-1, keepdims=True)

    denom = jnp.maximum(m, sigma) + 1e-6
    r = _MAXR / denom
    norm = jnp.floor(f * r)
    w = jnp.where(f >= thresh, norm - jnp.where(norm > 127.5, 256.0, 0.0),
                  0.0)

    out_ref[...] = jax.lax.dot_general(
        w, v, (((2,), (1,)), ((0,), (0,))),
        preferred_element_type=jnp.float32)


def kernel(x, Wk, Wq, Wv, gamma):
    b, t, d = x.shape
    g = jnp.reshape(gamma, (1, 1)).astype(jnp.float32)
    return pl.pallas_call(
        _head_body,
        grid=(b // _BB,),
        in_specs=[
            pl.BlockSpec((_BB, t, d), lambda i: (i, 0, 0)),
            pl.BlockSpec((d, d), lambda i: (0, 0)),
            pl.BlockSpec((d, d), lambda i: (0, 0)),
            pl.BlockSpec((d, d), lambda i: (0, 0)),
            pl.BlockSpec((t, t), lambda i: (0, 0)),
            pl.BlockSpec((1, 1), lambda i: (0, 0)),
        ],
        out_specs=pl.BlockSpec((_BB, t, d), lambda i: (i, 0, 0)),
        out_shape=jax.ShapeDtypeStruct((b, t, d), jnp.float32),
    )(x, Wq, Wk, Wv, jnp.asarray(_D8M), g)
